# Initial kernel scaffold; baseline (speedup 1.0000x reference)
import jax
import jax.numpy as jnp
from jax.experimental import pallas as pl


def kernel(feats, tags, seq_lengths, transitions):
    raise NotImplementedError("write your pallas kernel here")


# speedup vs baseline: 12.5394x; 12.5394x over previous
import jax
import jax.numpy as jnp
from jax.experimental import pallas as pl
from jax.experimental.pallas import tpu as pltpu

_TAGSET = 48
_T = 50
_START = 48
_STOP = 49
_B = 16
_L = 2048
_CHUNK = 256


def _forward_body(sl_ref, transT_ref, trans_ref, feats_ref, out_ref, fv_ref):
    pid = pl.program_id(0)

    @pl.when(pid == 0)
    def _init():
        lane = jax.lax.broadcasted_iota(jnp.int32, (_B, _T), 1)
        fv_ref[...] = jnp.where(lane == _START, 0.0, -10000.0)

    transT = transT_ref[...]
    mrow = jnp.max(transT, axis=0, keepdims=True)
    eT = jnp.exp(transT - mrow)
    sl = sl_ref[...]

    def step(i, fv):
        t = pid * _CHUNK + i
        feat = feats_ref[i]
        m = jnp.max(fv, axis=1, keepdims=True)
        e = jnp.exp(fv - m)
        s = jax.lax.dot_general(e, eT, (((1,), (0,)), ((), ())),
                                preferred_element_type=jnp.float32)
        nfv = m + mrow + jnp.log(s) + feat
        return jnp.where(t < sl, nfv, fv)

    fv = jax.lax.fori_loop(0, _CHUNK, step, fv_ref[...])
    fv_ref[...] = fv

    @pl.when(pid == pl.num_programs(0) - 1)
    def _final():
        term = fv + trans_ref[_STOP:_STOP + 1, :]
        m = jnp.max(term, axis=1, keepdims=True)
        out_ref[...] = m + jnp.log(
            jnp.sum(jnp.exp(term - m), axis=1, keepdims=True))


def _gold_body(sl_ref, transT_ref, feats_ref, tn_ref, tp_ref, out_ref):
    b = pl.program_id(0)

    @pl.when(b == 0)
    def _init():
        out_ref[...] = jnp.zeros_like(out_ref)

    sl = sl_ref[b, 0]
    f = feats_ref[0]
    tn = tn_ref[0]
    tp = tp_ref[0]
    srow = jax.lax.broadcasted_iota(jnp.int32, (_T, _L), 0)
    tcol = jax.lax.broadcasted_iota(jnp.int32, (1, _L), 1)
    maskr = (tcol < sl).astype(jnp.float32)
    ohn_raw = jnp.where(srow == tn, 1.0, 0.0)
    ohn = ohn_raw * maskr
    ohp = jnp.where(srow == tp, 1.0, 0.0)
    transT = transT_ref[...]
    R = jnp.dot(transT, ohn, preferred_element_type=jnp.float32)
    part = jnp.sum(f * ohn) + jnp.sum(R * ohp)
    ohlast = jnp.where(tcol == sl - 1, 1.0, 0.0)
    lastoh = jnp.sum(ohn_raw * ohlast, axis=1, keepdims=True)
    part = part + jnp.sum(transT[:, _STOP:_STOP + 1] * lastoh)
    out_ref[...] = out_ref[...] + part


def kernel(feats, tags, seq_lengths, transitions):
    featsT = jnp.transpose(feats, (1, 0, 2))
    featsTT = jnp.transpose(feats, (0, 2, 1))
    transT = jnp.transpose(transitions, (1, 0))
    sl_col = seq_lengths.reshape(_B, 1)
    tags3 = tags.reshape(_B, 1, _L)
    tags_prev = jnp.concatenate(
        [jnp.full((_B, 1), _START, dtype=tags.dtype), tags[:, :-1]], axis=1)
    tagsp3 = tags_prev.reshape(_B, 1, _L)

    n_blocks = _L // _CHUNK
    fs = pl.pallas_call(
        _forward_body,
        grid=(n_blocks,),
        in_specs=[
            pl.BlockSpec((_B, 1), lambda i: (0, 0)),
            pl.BlockSpec((_T, _T), lambda i: (0, 0)),
            pl.BlockSpec((_T, _T), lambda i: (0, 0)),
            pl.BlockSpec((_CHUNK, _B, _T), lambda i: (i, 0, 0)),
        ],
        out_specs=pl.BlockSpec((_B, 1), lambda i: (0, 0)),
        out_shape=jax.ShapeDtypeStruct((_B, 1), jnp.float32),
        scratch_shapes=[pltpu.VMEM((_B, _T), jnp.float32)],
        compiler_params=pltpu.CompilerParams(
            dimension_semantics=("arbitrary",)),
    )(sl_col, transT, transitions, featsT)

    gold = pl.pallas_call(
        _gold_body,
        grid=(_B,),
        in_specs=[
            pl.BlockSpec(memory_space=pltpu.SMEM),
            pl.BlockSpec((_T, _T), lambda b: (0, 0)),
            pl.BlockSpec((1, _T, _L), lambda b: (b, 0, 0)),
            pl.BlockSpec((1, 1, _L), lambda b: (b, 0, 0)),
            pl.BlockSpec((1, 1, _L), lambda b: (b, 0, 0)),
        ],
        out_specs=pl.BlockSpec((1, 1), lambda b: (0, 0)),
        out_shape=jax.ShapeDtypeStruct((1, 1), jnp.float32),
        compiler_params=pltpu.CompilerParams(
            dimension_semantics=("arbitrary",)),
    )(sl_col, transT, featsTT, tags3, tagsp3)

    return (jnp.sum(fs) - gold[0, 0]) / _B


# speedup vs baseline: 19.8551x; 1.5834x over previous
import jax
import jax.numpy as jnp
from jax.experimental import pallas as pl
from jax.experimental.pallas import tpu as pltpu

_TAGSET = 48
_T = 50
_START = 48
_STOP = 49
_B = 16
_L = 2048
_CHUNK = 256


_RENORM = 4


def _forward_body(sl_ref, transT_ref, trans_ref, feats_ref, out_ref,
                  a_ref, carry_ref, eg_ref):
    pid = pl.program_id(0)

    @pl.when(pid == 0)
    def _init():
        lane = jax.lax.broadcasted_iota(jnp.int32, (_B, _T), 1)
        a_ref[...] = jnp.where(lane == _START, 1.0, 0.0)
        carry_ref[...] = jnp.zeros((_B, 1), jnp.float32)

    transT = transT_ref[...]
    mrow = jnp.max(transT, axis=0, keepdims=True)
    eT = jnp.exp(transT - mrow)
    eg_ref[...] = jnp.exp(feats_ref[...] + mrow)
    sl = sl_ref[...]

    def group(g, ac):
        a, carry = ac
        for j in range(_RENORM):
            i = g * _RENORM + j
            t = pid * _CHUNK + i
            s = jax.lax.dot_general(a, eT, (((1,), (0,)), ((), ())),
                                    preferred_element_type=jnp.float32)
            s = s * eg_ref[i]
            a = jnp.where(t < sl, s, a)
        m = jnp.max(a, axis=1, keepdims=True)
        return a * (1.0 / m), carry + jnp.log(m)

    a, carry = jax.lax.fori_loop(0, _CHUNK // _RENORM, group,
                                 (a_ref[...], carry_ref[...]))
    a_ref[...] = a
    carry_ref[...] = carry

    @pl.when(pid == pl.num_programs(0) - 1)
    def _final():
        term = carry + jnp.log(a) + trans_ref[_STOP:_STOP + 1, :]
        m = jnp.max(term, axis=1, keepdims=True)
        out_ref[...] = m + jnp.log(
            jnp.sum(jnp.exp(term - m), axis=1, keepdims=True))


def _gold_body(sl_ref, transT_ref, feats_ref, tn_ref, tp_ref, out_ref):
    b = pl.program_id(0)

    @pl.when(b == 0)
    def _init():
        out_ref[...] = jnp.zeros_like(out_ref)

    sl = sl_ref[b, 0]
    f = feats_ref[0]
    tn = tn_ref[0]
    tp = tp_ref[0]
    srow = jax.lax.broadcasted_iota(jnp.int32, (_T, _L), 0)
    tcol = jax.lax.broadcasted_iota(jnp.int32, (1, _L), 1)
    maskr = (tcol < sl).astype(jnp.float32)
    ohn_raw = jnp.where(srow == tn, 1.0, 0.0)
    ohn = ohn_raw * maskr
    ohp = jnp.where(srow == tp, 1.0, 0.0)
    transT = transT_ref[...]
    R = jnp.dot(transT, ohn, preferred_element_type=jnp.float32)
    part = jnp.sum(f * ohn) + jnp.sum(R * ohp)
    ohlast = jnp.where(tcol == sl - 1, 1.0, 0.0)
    lastoh = jnp.sum(ohn_raw * ohlast, axis=1, keepdims=True)
    part = part + jnp.sum(transT[:, _STOP:_STOP + 1] * lastoh)
    out_ref[...] = out_ref[...] + part


def kernel(feats, tags, seq_lengths, transitions):
    featsT = jnp.transpose(feats, (1, 0, 2))
    featsTT = jnp.transpose(feats, (0, 2, 1))
    transT = jnp.transpose(transitions, (1, 0))
    sl_col = seq_lengths.reshape(_B, 1)
    tags3 = tags.reshape(_B, 1, _L)
    tags_prev = jnp.concatenate(
        [jnp.full((_B, 1), _START, dtype=tags.dtype), tags[:, :-1]], axis=1)
    tagsp3 = tags_prev.reshape(_B, 1, _L)

    n_blocks = _L // _CHUNK
    fs = pl.pallas_call(
        _forward_body,
        grid=(n_blocks,),
        in_specs=[
            pl.BlockSpec((_B, 1), lambda i: (0, 0)),
            pl.BlockSpec((_T, _T), lambda i: (0, 0)),
            pl.BlockSpec((_T, _T), lambda i: (0, 0)),
            pl.BlockSpec((_CHUNK, _B, _T), lambda i: (i, 0, 0)),
        ],
        out_specs=pl.BlockSpec((_B, 1), lambda i: (0, 0)),
        out_shape=jax.ShapeDtypeStruct((_B, 1), jnp.float32),
        scratch_shapes=[pltpu.VMEM((_B, _T), jnp.float32),
                        pltpu.VMEM((_B, 1), jnp.float32),
                        pltpu.VMEM((_CHUNK, _B, _T), jnp.float32)],
        compiler_params=pltpu.CompilerParams(
            dimension_semantics=("arbitrary",)),
    )(sl_col, transT, transitions, featsT)

    gold = pl.pallas_call(
        _gold_body,
        grid=(_B,),
        in_specs=[
            pl.BlockSpec(memory_space=pltpu.SMEM),
            pl.BlockSpec((_T, _T), lambda b: (0, 0)),
            pl.BlockSpec((1, _T, _L), lambda b: (b, 0, 0)),
            pl.BlockSpec((1, 1, _L), lambda b: (b, 0, 0)),
            pl.BlockSpec((1, 1, _L), lambda b: (b, 0, 0)),
        ],
        out_specs=pl.BlockSpec((1, 1), lambda b: (0, 0)),
        out_shape=jax.ShapeDtypeStruct((1, 1), jnp.float32),
        compiler_params=pltpu.CompilerParams(
            dimension_semantics=("arbitrary",)),
    )(sl_col, transT, featsTT, tags3, tagsp3)

    return (jnp.sum(fs) - gold[0, 0]) / _B


# speedup vs baseline: 30.7534x; 1.5489x over previous
import jax
import jax.numpy as jnp
from jax.experimental import pallas as pl
from jax.experimental.pallas import tpu as pltpu

_TAGSET = 48
_T = 50
_START = 48
_STOP = 49
_B = 16
_L = 2048
_CHUNK = 256


_RENORM = 4


def _forward_body(sl_ref, transT_ref, trans_ref, featsF_ref, featsR_ref,
                  out_ref, aF_ref, cF_ref, aB_ref, cB_ref, egF_ref, egB_ref):
    pid = pl.program_id(0)
    trans = trans_ref[...]
    transT = transT_ref[...]
    mrow = jnp.max(transT, axis=0, keepdims=True)
    eT = jnp.exp(transT - mrow)
    mcolB = jnp.max(trans, axis=0, keepdims=True)
    EB = jnp.exp(trans - mcolB)
    emcB = jnp.exp(mcolB)

    @pl.when(pid == 0)
    def _init():
        lane = jax.lax.broadcasted_iota(jnp.int32, (_B, _T), 1)
        aF_ref[...] = jnp.where(lane == _START, 1.0, 0.0)
        cF_ref[...] = jnp.zeros((_B, 1), jnp.float32)
        srow = trans[_STOP:_STOP + 1, :]
        m0 = jnp.max(srow, axis=1, keepdims=True)
        aB_ref[...] = jnp.broadcast_to(jnp.exp(srow - m0), (_B, _T))
        cB_ref[...] = jnp.broadcast_to(m0, (_B, 1))

    egF_ref[...] = jnp.exp(featsF_ref[...] + mrow)
    egB_ref[...] = jnp.exp(featsR_ref[...])
    sl = sl_ref[...]

    def group(g, st):
        aF, cF, aB, cB = st
        for j in range(_RENORM):
            i = g * _RENORM + j
            tF = pid * _CHUNK + i
            tB = _L - 1 - tF
            sF = jax.lax.dot_general(aF, eT, (((1,), (0,)), ((), ())),
                                     preferred_element_type=jnp.float32)
            sF = sF * egF_ref[i]
            aF = jnp.where(tF < sl, sF, aF)
            sB = jax.lax.dot_general(aB * egB_ref[i], EB,
                                     (((1,), (0,)), ((), ())),
                                     preferred_element_type=jnp.float32)
            sB = sB * emcB
            aB = jnp.where(tB < sl, sB, aB)
        mF = jnp.max(aF, axis=1, keepdims=True)
        mB = jnp.max(aB, axis=1, keepdims=True)
        return (aF * (1.0 / mF), cF + jnp.log(mF),
                aB * (1.0 / mB), cB + jnp.log(mB))

    aF, cF, aB, cB = jax.lax.fori_loop(
        0, _CHUNK // _RENORM, group,
        (aF_ref[...], cF_ref[...], aB_ref[...], cB_ref[...]))
    aF_ref[...] = aF
    cF_ref[...] = cF
    aB_ref[...] = aB
    cB_ref[...] = cB

    @pl.when(pid == pl.num_programs(0) - 1)
    def _final():
        d = jnp.sum(aF * aB, axis=1, keepdims=True)
        out_ref[...] = cF + cB + jnp.log(d)


def _gold_body(sl_ref, transT_ref, feats_ref, tn_ref, tp_ref, out_ref):
    b = pl.program_id(0)

    @pl.when(b == 0)
    def _init():
        out_ref[...] = jnp.zeros_like(out_ref)

    sl = sl_ref[b, 0]
    f = feats_ref[0]
    tn = tn_ref[0]
    tp = tp_ref[0]
    srow = jax.lax.broadcasted_iota(jnp.int32, (_T, _L), 0)
    tcol = jax.lax.broadcasted_iota(jnp.int32, (1, _L), 1)
    maskr = (tcol < sl).astype(jnp.float32)
    ohn_raw = jnp.where(srow == tn, 1.0, 0.0)
    ohn = ohn_raw * maskr
    ohp = jnp.where(srow == tp, 1.0, 0.0)
    transT = transT_ref[...]
    R = jnp.dot(transT, ohn, preferred_element_type=jnp.float32)
    part = jnp.sum(f * ohn) + jnp.sum(R * ohp)
    ohlast = jnp.where(tcol == sl - 1, 1.0, 0.0)
    lastoh = jnp.sum(ohn_raw * ohlast, axis=1, keepdims=True)
    part = part + jnp.sum(transT[:, _STOP:_STOP + 1] * lastoh)
    out_ref[...] = out_ref[...] + part


def kernel(feats, tags, seq_lengths, transitions):
    featsT = jnp.transpose(feats, (1, 0, 2))
    featsTT = jnp.transpose(feats, (0, 2, 1))
    transT = jnp.transpose(transitions, (1, 0))
    sl_col = seq_lengths.reshape(_B, 1)
    tags3 = tags.reshape(_B, 1, _L)
    tags_prev = jnp.concatenate(
        [jnp.full((_B, 1), _START, dtype=tags.dtype), tags[:, :-1]], axis=1)
    tagsp3 = tags_prev.reshape(_B, 1, _L)

    half = _L // 2
    featsF = featsT[:half]
    featsR = featsT[half:][::-1]
    n_blocks = half // _CHUNK
    fs = pl.pallas_call(
        _forward_body,
        grid=(n_blocks,),
        in_specs=[
            pl.BlockSpec((_B, 1), lambda i: (0, 0)),
            pl.BlockSpec((_T, _T), lambda i: (0, 0)),
            pl.BlockSpec((_T, _T), lambda i: (0, 0)),
            pl.BlockSpec((_CHUNK, _B, _T), lambda i: (i, 0, 0)),
            pl.BlockSpec((_CHUNK, _B, _T), lambda i: (i, 0, 0)),
        ],
        out_specs=pl.BlockSpec((_B, 1), lambda i: (0, 0)),
        out_shape=jax.ShapeDtypeStruct((_B, 1), jnp.float32),
        scratch_shapes=[pltpu.VMEM((_B, _T), jnp.float32),
                        pltpu.VMEM((_B, 1), jnp.float32),
                        pltpu.VMEM((_B, _T), jnp.float32),
                        pltpu.VMEM((_B, 1), jnp.float32),
                        pltpu.VMEM((_CHUNK, _B, _T), jnp.float32),
                        pltpu.VMEM((_CHUNK, _B, _T), jnp.float32)],
        compiler_params=pltpu.CompilerParams(
            dimension_semantics=("arbitrary",)),
    )(sl_col, transT, transitions, featsF, featsR)

    gold = pl.pallas_call(
        _gold_body,
        grid=(_B,),
        in_specs=[
            pl.BlockSpec(memory_space=pltpu.SMEM),
            pl.BlockSpec((_T, _T), lambda b: (0, 0)),
            pl.BlockSpec((1, _T, _L), lambda b: (b, 0, 0)),
            pl.BlockSpec((1, 1, _L), lambda b: (b, 0, 0)),
            pl.BlockSpec((1, 1, _L), lambda b: (b, 0, 0)),
        ],
        out_specs=pl.BlockSpec((1, 1), lambda b: (0, 0)),
        out_shape=jax.ShapeDtypeStruct((1, 1), jnp.float32),
        compiler_params=pltpu.CompilerParams(
            dimension_semantics=("arbitrary",)),
    )(sl_col, transT, featsTT, tags3, tagsp3)

    return (jnp.sum(fs) - gold[0, 0]) / _B


# speedup vs baseline: 33.5698x; 1.0916x over previous
import jax
import jax.numpy as jnp
from jax.experimental import pallas as pl
from jax.experimental.pallas import tpu as pltpu

_TAGSET = 48
_T = 50
_START = 48
_STOP = 49
_B = 16
_L = 2048
_CHUNK = 256
_RENORM = 4


def _forward_body(sl_ref, transT_ref, trans_ref, featsF_ref, featsR_ref,
                  out_ref, aF_ref, cF_ref, aB_ref, cB_ref, egF_ref, egB_ref):
    pid = pl.program_id(0)
    trans = trans_ref[...]
    transT = transT_ref[...]
    mrow = jnp.max(transT, axis=0, keepdims=True)
    eT = jnp.exp(transT - mrow)
    mcolB = jnp.max(trans, axis=0, keepdims=True)
    EB = jnp.exp(trans - mcolB)
    emcB = jnp.exp(mcolB)

    @pl.when(pid == 0)
    def _init():
        lane = jax.lax.broadcasted_iota(jnp.int32, (_B, _T), 1)
        aF_ref[...] = jnp.where(lane == _START, 1.0, 0.0)
        cF_ref[...] = jnp.zeros((_B, 1), jnp.float32)
        srow = trans[_STOP:_STOP + 1, :]
        m0 = jnp.max(srow, axis=1, keepdims=True)
        aB_ref[...] = jnp.broadcast_to(jnp.exp(srow - m0), (_B, _T))
        cB_ref[...] = jnp.broadcast_to(m0, (_B, 1))

    egF_ref[...] = jnp.exp(jnp.transpose(featsF_ref[...], (1, 0, 2)) + mrow)
    egB_ref[...] = jnp.exp(jnp.transpose(featsR_ref[...], (1, 0, 2)))
    sl = sl_ref[...]

    def group(g, st):
        aF, cF, aB, cB = st
        for j in range(_RENORM):
            i = g * _RENORM + j
            tF = pid * _CHUNK + i
            tB = _L - 1 - tF
            sF = jax.lax.dot_general(aF, eT, (((1,), (0,)), ((), ())),
                                     preferred_element_type=jnp.float32)
            sF = sF * egF_ref[i]
            aF = jnp.where(tF < sl, sF, aF)
            sB = jax.lax.dot_general(aB * egB_ref[_CHUNK - 1 - i], EB,
                                     (((1,), (0,)), ((), ())),
                                     preferred_element_type=jnp.float32)
            sB = sB * emcB
            aB = jnp.where(tB < sl, sB, aB)
        mF = jnp.max(aF, axis=1, keepdims=True)
        mB = jnp.max(aB, axis=1, keepdims=True)
        return (aF * (1.0 / mF), cF + jnp.log(mF),
                aB * (1.0 / mB), cB + jnp.log(mB))

    aF, cF, aB, cB = jax.lax.fori_loop(
        0, _CHUNK // _RENORM, group,
        (aF_ref[...], cF_ref[...], aB_ref[...], cB_ref[...]))
    aF_ref[...] = aF
    cF_ref[...] = cF
    aB_ref[...] = aB
    cB_ref[...] = cB

    @pl.when(pid == pl.num_programs(0) - 1)
    def _final():
        d = jnp.sum(aF * aB, axis=1, keepdims=True)
        out_ref[...] = cF + cB + jnp.log(d)


def _gold_body(sl_ref, trans_ref, feats_ref, tnT_ref, tpT_ref, out_ref):
    b = pl.program_id(0)

    @pl.when(b == 0)
    def _init():
        out_ref[...] = jnp.zeros_like(out_ref)

    sl = sl_ref[b, 0]
    f = feats_ref[0]
    bsel = jax.lax.broadcasted_iota(jnp.int32, (1, _B), 1) == b
    tn = jnp.sum(jnp.where(bsel, tnT_ref[...], 0), axis=1, keepdims=True)
    tp = jnp.sum(jnp.where(bsel, tpT_ref[...], 0), axis=1, keepdims=True)
    lane = jax.lax.broadcasted_iota(jnp.int32, (_L, _T), 1)
    rowi = jax.lax.broadcasted_iota(jnp.int32, (_L, 1), 0)
    maskc = (rowi < sl).astype(jnp.float32)
    ohn_raw = jnp.where(lane == tn, 1.0, 0.0)
    ohn = ohn_raw * maskc
    ohp = jnp.where(lane == tp, 1.0, 0.0)
    trans = trans_ref[...]
    R = jnp.dot(ohn, trans, preferred_element_type=jnp.float32)
    part = jnp.sum(f * ohn) + jnp.sum(R * ohp)
    lastc = (rowi == sl - 1).astype(jnp.float32)
    lastoh = jnp.sum(ohn_raw * lastc, axis=0, keepdims=True)
    part = part + jnp.sum(lastoh * trans[_STOP:_STOP + 1, :])
    out_ref[...] = out_ref[...] + part


def kernel(feats, tags, seq_lengths, transitions):
    transT = jnp.transpose(transitions, (1, 0))
    sl_col = seq_lengths.reshape(_B, 1)
    tagsT = jnp.transpose(tags, (1, 0))
    tags_prev = jnp.concatenate(
        [jnp.full((_B, 1), _START, dtype=tags.dtype), tags[:, :-1]], axis=1)
    tagspT = jnp.transpose(tags_prev, (1, 0))

    half = _L // 2
    n_blocks = half // _CHUNK
    nb_total = _L // _CHUNK
    fs = pl.pallas_call(
        _forward_body,
        grid=(n_blocks,),
        in_specs=[
            pl.BlockSpec((_B, 1), lambda i: (0, 0)),
            pl.BlockSpec((_T, _T), lambda i: (0, 0)),
            pl.BlockSpec((_T, _T), lambda i: (0, 0)),
            pl.BlockSpec((_B, _CHUNK, _T), lambda i: (0, i, 0)),
            pl.BlockSpec((_B, _CHUNK, _T), lambda i: (0, nb_total - 1 - i, 0)),
        ],
        out_specs=pl.BlockSpec((_B, 1), lambda i: (0, 0)),
        out_shape=jax.ShapeDtypeStruct((_B, 1), jnp.float32),
        scratch_shapes=[pltpu.VMEM((_B, _T), jnp.float32),
                        pltpu.VMEM((_B, 1), jnp.float32),
                        pltpu.VMEM((_B, _T), jnp.float32),
                        pltpu.VMEM((_B, 1), jnp.float32),
                        pltpu.VMEM((_CHUNK, _B, _T), jnp.float32),
                        pltpu.VMEM((_CHUNK, _B, _T), jnp.float32)],
        compiler_params=pltpu.CompilerParams(
            dimension_semantics=("arbitrary",)),
    )(sl_col, transT, transitions, feats, feats)

    gold = pl.pallas_call(
        _gold_body,
        grid=(_B,),
        in_specs=[
            pl.BlockSpec(memory_space=pltpu.SMEM),
            pl.BlockSpec((_T, _T), lambda b: (0, 0)),
            pl.BlockSpec((1, _L, _T), lambda b: (b, 0, 0)),
            pl.BlockSpec((_L, _B), lambda b: (0, 0)),
            pl.BlockSpec((_L, _B), lambda b: (0, 0)),
        ],
        out_specs=pl.BlockSpec((1, 1), lambda b: (0, 0)),
        out_shape=jax.ShapeDtypeStruct((1, 1), jnp.float32),
        compiler_params=pltpu.CompilerParams(
            dimension_semantics=("arbitrary",)),
    )(sl_col, transitions, feats, tagsT, tagspT)

    return (jnp.sum(fs) - gold[0, 0]) / _B
